# baseline (speedup 1.0000x reference)
import functools

import jax
import jax.numpy as jnp
from jax import lax
from jax.experimental import pallas as pl
from jax.experimental.pallas import tpu as pltpu
from jax.experimental.pallas import tpu_sc as plsc

E = 16
H = 1024
I = 1024
S = 2048
K = 2
NR = S * K
BLK = 256
MAX_TILES = NR // BLK + E
PAD = MAX_TILES * BLK

NC, NS, L = 2, 16, 16
NW = NC * NS
G_CH = 32
T_CH = 16

@functools.lru_cache(maxsize=None)
def _build_sc_kernels():
    mesh = plsc.VectorSubcoreMesh(
        core_axis_name="c", subcore_axis_name="s",
        num_cores=NC, num_subcores=NS)

    RPW = PAD // NW
    NCH = RPW // G_CH

    @functools.partial(
        pl.kernel,
        out_type=jax.ShapeDtypeStruct((PAD, H), jnp.float32),
        mesh=mesh,
        scratch_types=[
            pltpu.VMEM((RPW,), jnp.int32),
            pltpu.VMEM((G_CH, H), jnp.float32),
            pltpu.VMEM((G_CH, H), jnp.float32),
            pltpu.VMEM((G_CH, H), jnp.float32),
            pltpu.SemaphoreType.DMA,
            pltpu.SemaphoreType.DMA,
            pltpu.SemaphoreType.DMA,
            pltpu.SemaphoreType.DMA,
            pltpu.SemaphoreType.DMA,
            pltpu.SemaphoreType.DMA,
        ],
    )
    def sc_gather(x_hbm, tok_hbm, xs_hbm, idx_v, b0, b1, b2,
                  g0, g1, g2, s0, s1, s2):
        wid = lax.axis_index("s") * NC + lax.axis_index("c")
        base = wid * RPW
        bufs = (b0, b1, b2)
        gsems = (g0, g1, g2)
        ssems = (s0, s1, s2)
        pltpu.sync_copy(tok_hbm.at[pl.ds(base, RPW)], idx_v)

        def gfire(c):
            return pltpu.async_copy(
                x_hbm.at[idx_v.at[pl.ds(c * G_CH, G_CH)]],
                bufs[c % 3], gsems[c % 3])

        def sfire(c):
            return pltpu.async_copy(
                bufs[c % 3], xs_hbm.at[pl.ds(base + c * G_CH, G_CH)],
                ssems[c % 3])

        g = [None] * NCH
        s = [None] * NCH
        for c in range(min(3, NCH)):
            g[c] = gfire(c)
        for c in range(NCH):
            g[c].wait()
            s[c] = sfire(c)
            if c + 3 < NCH:
                s[c].wait()
                g[c + 3] = gfire(c + 3)
        for c in range(max(NCH - 3, 0), NCH):
            s[c].wait()

    TPW = S // NW
    TNCH = TPW // T_CH

    @functools.partial(
        pl.kernel,
        out_type=jax.ShapeDtypeStruct((S, H), jnp.float32),
        mesh=mesh,
        scratch_types=[
            pltpu.VMEM((TPW,), jnp.int32),
            pltpu.VMEM((TPW,), jnp.int32),
            pltpu.VMEM((T_CH, H), jnp.float32),
            pltpu.VMEM((T_CH, H), jnp.float32),
            pltpu.VMEM((T_CH, H), jnp.float32),
            pltpu.VMEM((T_CH, H), jnp.float32),
            pltpu.VMEM((T_CH, H), jnp.float32),
            pltpu.VMEM((T_CH, H), jnp.float32),
            pltpu.SemaphoreType.DMA,
            pltpu.SemaphoreType.DMA,
            pltpu.SemaphoreType.DMA,
            pltpu.SemaphoreType.DMA,
        ],
    )
    def sc_combine(ys_hbm, p0_hbm, p1_hbm, sh_hbm, out_hbm,
                   i0_v, i1_v, y0a, y1a, sha, y0b, y1b, shb,
                   ga, gb, sa, sb):
        wid = lax.axis_index("s") * NC + lax.axis_index("c")
        base = wid * TPW
        y0s = (y0a, y0b)
        y1s = (y1a, y1b)
        shs = (sha, shb)
        gsems = (ga, gb)
        ssems = (sa, sb)
        pltpu.sync_copy(p0_hbm.at[pl.ds(base, TPW)], i0_v)
        pltpu.sync_copy(p1_hbm.at[pl.ds(base, TPW)], i1_v)

        def fire_in(c):
            sl = c % 2
            return (
                pltpu.async_copy(
                    ys_hbm.at[i0_v.at[pl.ds(c * T_CH, T_CH)]],
                    y0s[sl], gsems[sl]),
                pltpu.async_copy(
                    ys_hbm.at[i1_v.at[pl.ds(c * T_CH, T_CH)]],
                    y1s[sl], gsems[sl]),
                pltpu.async_copy(
                    sh_hbm.at[pl.ds(base + c * T_CH, T_CH)],
                    shs[sl], gsems[sl]),
            )

        def fire_out(c):
            sl = c % 2
            return pltpu.async_copy(
                y0s[sl], out_hbm.at[pl.ds(base + c * T_CH, T_CH)], ssems[sl])

        ins = [None] * TNCH
        outs = [None] * TNCH
        for c in range(min(2, TNCH)):
            ins[c] = fire_in(c)
        for c in range(TNCH):
            sl = c % 2
            for cp in ins[c]:
                cp.wait()
            y0r, y1r, shr = y0s[sl], y1s[sl], shs[sl]

            def row_body(r, _):
                for cc in range(H // L):
                    sli = pl.ds(cc * L, L)
                    y0r[r, sli] = y0r[r, sli] + y1r[r, sli] + shr[r, sli]
                return 0

            lax.fori_loop(0, T_CH, row_body, 0)
            outs[c] = fire_out(c)
            if c + 2 < TNCH:
                outs[c].wait()
                ins[c + 2] = fire_in(c + 2)
        for c in range(max(TNCH - 2, 0), TNCH):
            outs[c].wait()

    return sc_gather, sc_combine


def _expert_body(te_ref, tv_ref, x_ref, w1_ref, b1_ref, w2_ref, b2_ref,
                 sw_ref, y_ref, h_ref):
    t = pl.program_id(0)
    n = pl.program_id(1)

    @pl.when((tv_ref[t] > 0) & (n == 0))
    def _():
        x = x_ref[...]
        h_ref[...] = (jnp.dot(x, w1_ref[0],
                              preferred_element_type=jnp.float32)
                      + b1_ref[0])

    @pl.when((tv_ref[t] > 0) & (n == 1))
    def _():
        x = x_ref[...]
        cur = (jnp.dot(x, w1_ref[0], preferred_element_type=jnp.float32)
               + b1_ref[0])
        prev = h_ref[...]
        odd = (t % 2) == 1
        a = jnp.where(odd, cur, prev)
        b = jnp.where(odd, prev, cur)
        hh = (a * jax.nn.sigmoid(a)) * b
        y = jnp.dot(hh, w2_ref[0], preferred_element_type=jnp.float32)
        y = y + b2_ref[0]
        y_ref[...] = y * sw_ref[...]


def _run_experts(tile_expert, tile_valid, xs, w1, b1, w2, b2, srw):
    grid_spec = pltpu.PrefetchScalarGridSpec(
        num_scalar_prefetch=2,
        grid=(MAX_TILES, 2),
        in_specs=[
            pl.BlockSpec((BLK, H), lambda t, n, te, tv: (t, 0)),
            pl.BlockSpec((1, H, I),
                         lambda t, n, te, tv: (te[t], 0, (n + t) % 2)),
            pl.BlockSpec((1, 1, I),
                         lambda t, n, te, tv: (te[t], 0, (n + t) % 2)),
            pl.BlockSpec((1, I, H), lambda t, n, te, tv: (te[t], 0, 0)),
            pl.BlockSpec((1, 1, H), lambda t, n, te, tv: (te[t], 0, 0)),
            pl.BlockSpec((BLK, 1), lambda t, n, te, tv: (t, 0)),
        ],
        out_specs=pl.BlockSpec((BLK, H), lambda t, n, te, tv: (t, 0)),
        scratch_shapes=[pltpu.VMEM((BLK, I), jnp.float32)],
    )
    return pl.pallas_call(
        _expert_body,
        grid_spec=grid_spec,
        out_shape=jax.ShapeDtypeStruct((PAD, H), jnp.float32),
        compiler_params=pltpu.CompilerParams(
            dimension_semantics=("arbitrary", "arbitrary")),
    )(tile_expert, tile_valid, xs, w1, b1.reshape(E, 1, 2 * I), w2,
      b2.reshape(E, 1, H), srw)


def _shared_body(x_ref, w1_ref, b1_ref, w2_ref, b2_ref, o_ref):
    x = x_ref[...]
    h = jnp.dot(x, w1_ref[...], preferred_element_type=jnp.float32)
    h = h + b1_ref[...]
    a = h[:, :I]
    b = h[:, I:]
    hh = (a * jax.nn.sigmoid(a)) * b
    o = jnp.dot(hh, w2_ref[...], preferred_element_type=jnp.float32)
    o_ref[...] = o + b2_ref[...]


def _run_shared(x, sw1, sb1, sw2, sb2):
    nblk = S // BLK
    return pl.pallas_call(
        _shared_body,
        grid=(nblk,),
        in_specs=[
            pl.BlockSpec((BLK, H), lambda t: (t, 0)),
            pl.BlockSpec((H, 2 * I), lambda t: (0, 0)),
            pl.BlockSpec((1, 2 * I), lambda t: (0, 0)),
            pl.BlockSpec((I, H), lambda t: (0, 0)),
            pl.BlockSpec((1, H), lambda t: (0, 0)),
        ],
        out_specs=pl.BlockSpec((BLK, H), lambda t: (t, 0)),
        out_shape=jax.ShapeDtypeStruct((S, H), jnp.float32),
        compiler_params=pltpu.CompilerParams(
            dimension_semantics=("arbitrary",)),
    )(x, sw1, sb1.reshape(1, 2 * I), sw2, sb2.reshape(1, H))


_RR = 32
_RL = 128


def _route_body(idx_ref, pos_ref, te_ref, tv_ref):
    idx = idx_ref[...]
    ii = lax.broadcasted_iota(jnp.int32, (_RL, _RL), 0)
    jj = lax.broadcasted_iota(jnp.int32, (_RL, _RL), 1)
    upp = (ii <= jj).astype(jnp.float32)
    i2 = lax.broadcasted_iota(jnp.int32, (_RR, _RR), 0)
    j2 = lax.broadcasted_iota(jnp.int32, (_RR, _RR), 1)
    lstrict = (j2 < i2).astype(jnp.float32)

    masks = []
    rexs = []
    sizes = []
    for e in range(E):
        m = (idx == e).astype(jnp.float32)
        cr = jnp.dot(m, upp, preferred_element_type=jnp.float32)
        tot = cr[:, _RL - 1:_RL]
        prior = jnp.dot(lstrict, tot, preferred_element_type=jnp.float32)
        rexs.append(cr - m + prior)
        masks.append(m)
        sizes.append(jnp.sum(tot).astype(jnp.int32))

    tile_cum = []
    c = jnp.int32(0)
    aligned = []
    for e in range(E):
        nt = (sizes[e] + BLK - 1) // BLK
        aligned.append((c * BLK).astype(jnp.float32))
        c = c + nt
        tile_cum.append(c)

    pos = jnp.zeros((_RR, _RL), jnp.float32)
    for e in range(E):
        pos = pos + masks[e] * (rexs[e] + aligned[e])
    pos_ref[...] = pos.astype(jnp.int32)

    ti = lax.broadcasted_iota(jnp.int32, (1, _RL), 1)
    te = jnp.zeros((1, _RL), jnp.int32)
    for e in range(E):
        te = te + (tile_cum[e] <= ti).astype(jnp.int32)
    te_ref[...] = jnp.minimum(te, E - 1)
    tv_ref[...] = (ti < tile_cum[E - 1]).astype(jnp.int32)


def _routing(flat_idx, flat_w):
    pos2, te, tv = pl.pallas_call(
        _route_body,
        out_shape=(
            jax.ShapeDtypeStruct((_RR, _RL), jnp.int32),
            jax.ShapeDtypeStruct((1, _RL), jnp.int32),
            jax.ShapeDtypeStruct((1, _RL), jnp.int32),
        ),
    )(flat_idx.reshape(_RR, _RL))
    pos = pos2.reshape(NR)
    tile_expert = te.reshape(_RL)
    tile_valid = tv.reshape(_RL)

    r_ar = jnp.arange(NR, dtype=jnp.int32)
    tok_src = (jnp.arange(PAD, dtype=jnp.int32) % S).at[pos].set(r_ar // K)
    srw = jnp.zeros((PAD,), jnp.float32).at[pos].set(flat_w)
    p0 = pos[0::2]
    p1 = pos[1::2]
    return tile_expert, tile_valid, tok_src, srw, p0, p1


def kernel(hidden_states, topk_weight, topk_idx, w1, b1, w2, b2,
           sw1, sb1, sw2, sb2):
    orig_shape = hidden_states.shape
    x = hidden_states.reshape(S, H)
    flat_idx = topk_idx.reshape(NR).astype(jnp.int32)
    flat_w = topk_weight.reshape(NR)
    tile_expert, tile_valid, tok_src, srw, p0, p1 = _routing(flat_idx, flat_w)

    sc_gather, sc_combine = _build_sc_kernels()
    xs = sc_gather(x, tok_src)
    ys = _run_experts(tile_expert, tile_valid, xs, w1, b1, w2, b2,
                      srw.reshape(PAD, 1))
    sh = _run_shared(x, sw1, sb1, sw2, sb2)
    out = sc_combine(ys, p0, p1, sh)
    return out.reshape(orig_shape)


# speedup vs baseline: 1.1130x; 1.1130x over previous
import functools

import jax
import jax.numpy as jnp
from jax import lax
from jax.experimental import pallas as pl
from jax.experimental.pallas import tpu as pltpu
from jax.experimental.pallas import tpu_sc as plsc

E = 16
H = 1024
I = 1024
S = 2048
K = 2
NR = S * K
BLK = 256
MAX_TILES = NR // BLK + E
PAD = MAX_TILES * BLK

NC, NS, L = 2, 16, 16
NW = NC * NS
G_CH = 32
T_CH = 16

@functools.lru_cache(maxsize=None)
def _build_sc_kernels():
    mesh = plsc.VectorSubcoreMesh(
        core_axis_name="c", subcore_axis_name="s",
        num_cores=NC, num_subcores=NS)

    RPW = PAD // NW
    NCH = RPW // G_CH

    @functools.partial(
        pl.kernel,
        out_type=jax.ShapeDtypeStruct((PAD, H), jnp.float32),
        mesh=mesh,
        scratch_types=[
            pltpu.VMEM((RPW,), jnp.int32),
            pltpu.VMEM((G_CH, H), jnp.float32),
            pltpu.VMEM((G_CH, H), jnp.float32),
            pltpu.VMEM((G_CH, H), jnp.float32),
            pltpu.SemaphoreType.DMA,
            pltpu.SemaphoreType.DMA,
            pltpu.SemaphoreType.DMA,
            pltpu.SemaphoreType.DMA,
            pltpu.SemaphoreType.DMA,
            pltpu.SemaphoreType.DMA,
        ],
    )
    def sc_gather(x_hbm, tok_hbm, xs_hbm, idx_v, b0, b1, b2,
                  g0, g1, g2, s0, s1, s2):
        wid = lax.axis_index("s") * NC + lax.axis_index("c")
        base = wid * RPW
        bufs = (b0, b1, b2)
        gsems = (g0, g1, g2)
        ssems = (s0, s1, s2)
        pltpu.sync_copy(tok_hbm.at[pl.ds(base, RPW)], idx_v)

        def gfire(c):
            return pltpu.async_copy(
                x_hbm.at[idx_v.at[pl.ds(c * G_CH, G_CH)]],
                bufs[c % 3], gsems[c % 3])

        def sfire(c):
            return pltpu.async_copy(
                bufs[c % 3], xs_hbm.at[pl.ds(base + c * G_CH, G_CH)],
                ssems[c % 3])

        g = [None] * NCH
        s = [None] * NCH
        for c in range(min(3, NCH)):
            g[c] = gfire(c)
        for c in range(NCH):
            g[c].wait()
            s[c] = sfire(c)
            if c + 3 < NCH:
                s[c].wait()
                g[c + 3] = gfire(c + 3)
        for c in range(max(NCH - 3, 0), NCH):
            s[c].wait()

    TPW = S // NW
    TNCH = TPW // T_CH

    @functools.partial(
        pl.kernel,
        out_type=jax.ShapeDtypeStruct((S, H), jnp.float32),
        mesh=mesh,
        scratch_types=[
            pltpu.VMEM((TPW,), jnp.int32),
            pltpu.VMEM((TPW,), jnp.int32),
            pltpu.VMEM((T_CH, H), jnp.float32),
            pltpu.VMEM((T_CH, H), jnp.float32),
            pltpu.VMEM((T_CH, H), jnp.float32),
            pltpu.VMEM((T_CH, H), jnp.float32),
            pltpu.VMEM((T_CH, H), jnp.float32),
            pltpu.VMEM((T_CH, H), jnp.float32),
            pltpu.SemaphoreType.DMA,
            pltpu.SemaphoreType.DMA,
            pltpu.SemaphoreType.DMA,
            pltpu.SemaphoreType.DMA,
        ],
    )
    def sc_combine(ys_hbm, p0_hbm, p1_hbm, sh_hbm, out_hbm,
                   i0_v, i1_v, y0a, y1a, sha, y0b, y1b, shb,
                   ga, gb, sa, sb):
        wid = lax.axis_index("s") * NC + lax.axis_index("c")
        base = wid * TPW
        y0s = (y0a, y0b)
        y1s = (y1a, y1b)
        shs = (sha, shb)
        gsems = (ga, gb)
        ssems = (sa, sb)
        pltpu.sync_copy(p0_hbm.at[pl.ds(base, TPW)], i0_v)
        pltpu.sync_copy(p1_hbm.at[pl.ds(base, TPW)], i1_v)

        def fire_in(c):
            sl = c % 2
            return (
                pltpu.async_copy(
                    ys_hbm.at[i0_v.at[pl.ds(c * T_CH, T_CH)]],
                    y0s[sl], gsems[sl]),
                pltpu.async_copy(
                    ys_hbm.at[i1_v.at[pl.ds(c * T_CH, T_CH)]],
                    y1s[sl], gsems[sl]),
                pltpu.async_copy(
                    sh_hbm.at[pl.ds(base + c * T_CH, T_CH)],
                    shs[sl], gsems[sl]),
            )

        def fire_out(c):
            sl = c % 2
            return pltpu.async_copy(
                y0s[sl], out_hbm.at[pl.ds(base + c * T_CH, T_CH)], ssems[sl])

        ins = [None] * TNCH
        outs = [None] * TNCH
        for c in range(min(2, TNCH)):
            ins[c] = fire_in(c)
        for c in range(TNCH):
            sl = c % 2
            for cp in ins[c]:
                cp.wait()
            y0r, y1r, shr = y0s[sl], y1s[sl], shs[sl]

            def row_body(r, _):
                for cc in range(H // L):
                    sli = pl.ds(cc * L, L)
                    y0r[r, sli] = y0r[r, sli] + y1r[r, sli] + shr[r, sli]
                return 0

            lax.fori_loop(0, T_CH, row_body, 0)
            outs[c] = fire_out(c)
            if c + 2 < TNCH:
                outs[c].wait()
                ins[c + 2] = fire_in(c + 2)
        for c in range(max(TNCH - 2, 0), TNCH):
            outs[c].wait()

    return sc_gather, sc_combine


def _expert_body(te_ref, tv_ref, x_ref, w1_ref, b1_ref, w2_ref, b2_ref,
                 sw_ref, y_ref):
    t = pl.program_id(0)

    @pl.when(tv_ref[t] > 0)
    def _():
        x = x_ref[...]
        h = jnp.dot(x, w1_ref[0], preferred_element_type=jnp.float32)
        h = h + b1_ref[0]
        a = h[:, :I]
        b = h[:, I:]
        hh = (a * jax.nn.sigmoid(a)) * b
        y = jnp.dot(hh, w2_ref[0], preferred_element_type=jnp.float32)
        y = y + b2_ref[0]
        y_ref[...] = y * sw_ref[...]


def _run_experts(tile_expert, tile_valid, xs, w1, b1, w2, b2, srw):
    grid_spec = pltpu.PrefetchScalarGridSpec(
        num_scalar_prefetch=2,
        grid=(MAX_TILES,),
        in_specs=[
            pl.BlockSpec((BLK, H), lambda t, te, tv: (t, 0)),
            pl.BlockSpec((1, H, 2 * I), lambda t, te, tv: (te[t], 0, 0)),
            pl.BlockSpec((1, 1, 2 * I), lambda t, te, tv: (te[t], 0, 0)),
            pl.BlockSpec((1, I, H), lambda t, te, tv: (te[t], 0, 0)),
            pl.BlockSpec((1, 1, H), lambda t, te, tv: (te[t], 0, 0)),
            pl.BlockSpec((BLK, 1), lambda t, te, tv: (t, 0)),
        ],
        out_specs=pl.BlockSpec((BLK, H), lambda t, te, tv: (t, 0)),
    )
    return pl.pallas_call(
        _expert_body,
        grid_spec=grid_spec,
        out_shape=jax.ShapeDtypeStruct((PAD, H), jnp.float32),
        compiler_params=pltpu.CompilerParams(
            dimension_semantics=("arbitrary",)),
    )(tile_expert, tile_valid, xs, w1, b1.reshape(E, 1, 2 * I), w2,
      b2.reshape(E, 1, H), srw)


def _shared_body(x_ref, w1_ref, b1_ref, w2_ref, b2_ref, o_ref):
    x = x_ref[...]
    h = jnp.dot(x, w1_ref[...], preferred_element_type=jnp.float32)
    h = h + b1_ref[...]
    a = h[:, :I]
    b = h[:, I:]
    hh = (a * jax.nn.sigmoid(a)) * b
    o = jnp.dot(hh, w2_ref[...], preferred_element_type=jnp.float32)
    o_ref[...] = o + b2_ref[...]


def _run_shared(x, sw1, sb1, sw2, sb2):
    nblk = S // BLK
    return pl.pallas_call(
        _shared_body,
        grid=(nblk,),
        in_specs=[
            pl.BlockSpec((BLK, H), lambda t: (t, 0)),
            pl.BlockSpec((H, 2 * I), lambda t: (0, 0)),
            pl.BlockSpec((1, 2 * I), lambda t: (0, 0)),
            pl.BlockSpec((I, H), lambda t: (0, 0)),
            pl.BlockSpec((1, H), lambda t: (0, 0)),
        ],
        out_specs=pl.BlockSpec((BLK, H), lambda t: (t, 0)),
        out_shape=jax.ShapeDtypeStruct((S, H), jnp.float32),
        compiler_params=pltpu.CompilerParams(
            dimension_semantics=("arbitrary",)),
    )(x, sw1, sb1.reshape(1, 2 * I), sw2, sb2.reshape(1, H))


_RR = 32
_RL = 128


def _route_body(idx_ref, pos_ref, te_ref, tv_ref):
    idx = idx_ref[...]
    ii = lax.broadcasted_iota(jnp.int32, (_RL, _RL), 0)
    jj = lax.broadcasted_iota(jnp.int32, (_RL, _RL), 1)
    upp = (ii <= jj).astype(jnp.float32)
    i2 = lax.broadcasted_iota(jnp.int32, (_RR, _RR), 0)
    j2 = lax.broadcasted_iota(jnp.int32, (_RR, _RR), 1)
    lstrict = (j2 < i2).astype(jnp.float32)

    masks = []
    rexs = []
    sizes = []
    for e in range(E):
        m = (idx == e).astype(jnp.float32)
        cr = jnp.dot(m, upp, preferred_element_type=jnp.float32)
        tot = cr[:, _RL - 1:_RL]
        prior = jnp.dot(lstrict, tot, preferred_element_type=jnp.float32)
        rexs.append(cr - m + prior)
        masks.append(m)
        sizes.append(jnp.sum(tot).astype(jnp.int32))

    tile_cum = []
    c = jnp.int32(0)
    aligned = []
    for e in range(E):
        nt = (sizes[e] + BLK - 1) // BLK
        aligned.append((c * BLK).astype(jnp.float32))
        c = c + nt
        tile_cum.append(c)

    pos = jnp.zeros((_RR, _RL), jnp.float32)
    for e in range(E):
        pos = pos + masks[e] * (rexs[e] + aligned[e])
    pos_ref[...] = pos.astype(jnp.int32)

    ti = lax.broadcasted_iota(jnp.int32, (1, _RL), 1)
    te = jnp.zeros((1, _RL), jnp.int32)
    for e in range(E):
        te = te + (tile_cum[e] <= ti).astype(jnp.int32)
    te_ref[...] = jnp.minimum(te, E - 1)
    tv_ref[...] = (ti < tile_cum[E - 1]).astype(jnp.int32)


def _routing(flat_idx, flat_w):
    pos2, te, tv = pl.pallas_call(
        _route_body,
        out_shape=(
            jax.ShapeDtypeStruct((_RR, _RL), jnp.int32),
            jax.ShapeDtypeStruct((1, _RL), jnp.int32),
            jax.ShapeDtypeStruct((1, _RL), jnp.int32),
        ),
    )(flat_idx.reshape(_RR, _RL))
    pos = pos2.reshape(NR)
    tile_expert = te.reshape(_RL)
    tile_valid = tv.reshape(_RL)

    r_ar = jnp.arange(NR, dtype=jnp.int32)
    tok_src = (jnp.arange(PAD, dtype=jnp.int32) % S).at[pos].set(r_ar // K)
    srw = jnp.zeros((PAD,), jnp.float32).at[pos].set(flat_w)
    p0 = pos[0::2]
    p1 = pos[1::2]
    return tile_expert, tile_valid, tok_src, srw, p0, p1


def kernel(hidden_states, topk_weight, topk_idx, w1, b1, w2, b2,
           sw1, sb1, sw2, sb2):
    orig_shape = hidden_states.shape
    x = hidden_states.reshape(S, H)
    flat_idx = topk_idx.reshape(NR).astype(jnp.int32)
    flat_w = topk_weight.reshape(NR)
    tile_expert, tile_valid, tok_src, srw, p0, p1 = _routing(flat_idx, flat_w)

    sc_gather, sc_combine = _build_sc_kernels()
    sh = _run_shared(x, sw1, sb1, sw2, sb2)
    xs = sc_gather(x, tok_src)
    ys = _run_experts(tile_expert, tile_valid, xs, w1, b1, w2, b2,
                      srw.reshape(PAD, 1))
    out = sc_combine(ys, p0, p1, sh)
    return out.reshape(orig_shape)


# speedup vs baseline: 1.1774x; 1.0579x over previous
import functools

import jax
import jax.numpy as jnp
from jax import lax
from jax.experimental import pallas as pl
from jax.experimental.pallas import tpu as pltpu
from jax.experimental.pallas import tpu_sc as plsc

E = 16
H = 1024
I = 1024
S = 2048
K = 2
NR = S * K
BLK = 256
MAX_TILES = NR // BLK + E
PAD = MAX_TILES * BLK

NC, NS, L = 2, 16, 16
NW = NC * NS
G_CH = 32
T_CH = 16

@functools.lru_cache(maxsize=None)
def _build_sc_kernels():
    mesh = plsc.VectorSubcoreMesh(
        core_axis_name="c", subcore_axis_name="s",
        num_cores=NC, num_subcores=NS)

    RPW = PAD // NW
    NCH = RPW // G_CH

    @functools.partial(
        pl.kernel,
        out_type=jax.ShapeDtypeStruct((PAD, H), jnp.float32),
        mesh=mesh,
        scratch_types=[
            pltpu.VMEM((RPW,), jnp.int32),
            pltpu.VMEM((G_CH, H), jnp.float32),
            pltpu.VMEM((G_CH, H), jnp.float32),
            pltpu.VMEM((G_CH, H), jnp.float32),
            pltpu.SemaphoreType.DMA,
            pltpu.SemaphoreType.DMA,
            pltpu.SemaphoreType.DMA,
            pltpu.SemaphoreType.DMA,
            pltpu.SemaphoreType.DMA,
            pltpu.SemaphoreType.DMA,
        ],
    )
    def sc_gather(x_hbm, tok_hbm, xs_hbm, idx_v, b0, b1, b2,
                  g0, g1, g2, s0, s1, s2):
        wid = lax.axis_index("s") * NC + lax.axis_index("c")
        base = wid * RPW
        bufs = (b0, b1, b2)
        gsems = (g0, g1, g2)
        ssems = (s0, s1, s2)
        pltpu.sync_copy(tok_hbm.at[pl.ds(base, RPW)], idx_v)

        def gfire(c):
            return pltpu.async_copy(
                x_hbm.at[idx_v.at[pl.ds(c * G_CH, G_CH)]],
                bufs[c % 3], gsems[c % 3])

        def sfire(c):
            return pltpu.async_copy(
                bufs[c % 3], xs_hbm.at[pl.ds(base + c * G_CH, G_CH)],
                ssems[c % 3])

        g = [None] * NCH
        s = [None] * NCH
        for c in range(min(3, NCH)):
            g[c] = gfire(c)
        for c in range(NCH):
            g[c].wait()
            s[c] = sfire(c)
            if c + 3 < NCH:
                s[c].wait()
                g[c + 3] = gfire(c + 3)
        for c in range(max(NCH - 3, 0), NCH):
            s[c].wait()

    TPW = S // NW
    TNCH = TPW // T_CH

    @functools.partial(
        pl.kernel,
        out_type=jax.ShapeDtypeStruct((S, H), jnp.float32),
        mesh=mesh,
        scratch_types=[
            pltpu.VMEM((TPW,), jnp.int32),
            pltpu.VMEM((TPW,), jnp.int32),
            pltpu.VMEM((T_CH, H), jnp.float32),
            pltpu.VMEM((T_CH, H), jnp.float32),
            pltpu.VMEM((T_CH, H), jnp.float32),
            pltpu.VMEM((T_CH, H), jnp.float32),
            pltpu.VMEM((T_CH, H), jnp.float32),
            pltpu.VMEM((T_CH, H), jnp.float32),
            pltpu.SemaphoreType.DMA,
            pltpu.SemaphoreType.DMA,
            pltpu.SemaphoreType.DMA,
            pltpu.SemaphoreType.DMA,
        ],
    )
    def sc_combine(ys_hbm, p0_hbm, p1_hbm, sh_hbm, out_hbm,
                   i0_v, i1_v, y0a, y1a, sha, y0b, y1b, shb,
                   ga, gb, sa, sb):
        wid = lax.axis_index("s") * NC + lax.axis_index("c")
        base = wid * TPW
        y0s = (y0a, y0b)
        y1s = (y1a, y1b)
        shs = (sha, shb)
        gsems = (ga, gb)
        ssems = (sa, sb)
        pltpu.sync_copy(p0_hbm.at[pl.ds(base, TPW)], i0_v)
        pltpu.sync_copy(p1_hbm.at[pl.ds(base, TPW)], i1_v)

        def fire_in(c):
            sl = c % 2
            return (
                pltpu.async_copy(
                    ys_hbm.at[i0_v.at[pl.ds(c * T_CH, T_CH)]],
                    y0s[sl], gsems[sl]),
                pltpu.async_copy(
                    ys_hbm.at[i1_v.at[pl.ds(c * T_CH, T_CH)]],
                    y1s[sl], gsems[sl]),
                pltpu.async_copy(
                    sh_hbm.at[pl.ds(base + c * T_CH, T_CH)],
                    shs[sl], gsems[sl]),
            )

        def fire_out(c):
            sl = c % 2
            return pltpu.async_copy(
                y0s[sl], out_hbm.at[pl.ds(base + c * T_CH, T_CH)], ssems[sl])

        ins = [None] * TNCH
        outs = [None] * TNCH
        for c in range(min(2, TNCH)):
            ins[c] = fire_in(c)
        for c in range(TNCH):
            sl = c % 2
            for cp in ins[c]:
                cp.wait()
            y0r, y1r, shr = y0s[sl], y1s[sl], shs[sl]

            def row_body(r, _):
                for cc in range(H // L):
                    sli = pl.ds(cc * L, L)
                    y0r[r, sli] = y0r[r, sli] + y1r[r, sli] + shr[r, sli]
                return 0

            lax.fori_loop(0, T_CH, row_body, 0)
            outs[c] = fire_out(c)
            if c + 2 < TNCH:
                outs[c].wait()
                ins[c + 2] = fire_in(c + 2)
        for c in range(max(TNCH - 2, 0), TNCH):
            outs[c].wait()

    return sc_gather, sc_combine


def _expert_body(te_ref, tv_ref, fi_ref, sl_ref, nx_ref, is_ref,
                 x_ref, w1_any, b1_ref, w2_any, b2_ref, sw_ref, y_ref,
                 w1b, w2b, sem1, sem2):
    t = pl.program_id(0)
    slot = sl_ref[t]
    e = te_ref[t]

    def w1_copy(src_e, dst_slot):
        return pltpu.make_async_copy(
            w1_any.at[pl.ds(src_e, 1)], w1b.at[pl.ds(dst_slot, 1)],
            sem1.at[dst_slot])

    def w2_copy(src_e, dst_slot):
        return pltpu.make_async_copy(
            w2_any.at[pl.ds(src_e, 1)], w2b.at[pl.ds(dst_slot, 1)],
            sem2.at[dst_slot])

    @pl.when(t == 0)
    def _():
        w1_copy(e, slot).start()
        w2_copy(e, slot).start()

    @pl.when(fi_ref[t] == 1)
    def _():
        w1_copy(e, slot).wait()
        w2_copy(e, slot).wait()

    @pl.when(is_ref[t] == 1)
    def _():
        w1_copy(nx_ref[t], 1 - slot).start()
        w2_copy(nx_ref[t], 1 - slot).start()

    @pl.when(tv_ref[t] > 0)
    def _():
        x = x_ref[...]
        h = jnp.dot(x, w1b[slot], preferred_element_type=jnp.float32)
        h = h + b1_ref[0]
        a = h[:, :I]
        b = h[:, I:]
        hh = (a * jax.nn.sigmoid(a)) * b
        y = jnp.dot(hh, w2b[slot], preferred_element_type=jnp.float32)
        y = y + b2_ref[0]
        y_ref[...] = y * sw_ref[...]


def _run_experts(tile_expert, tile_valid, sched, xs, w1, b1, w2, b2, srw):
    fi, sl, nx, isu = sched
    grid_spec = pltpu.PrefetchScalarGridSpec(
        num_scalar_prefetch=6,
        grid=(MAX_TILES,),
        in_specs=[
            pl.BlockSpec((BLK, H), lambda t, *_: (t, 0)),
            pl.BlockSpec(memory_space=pl.ANY),
            pl.BlockSpec((1, 1, 2 * I), lambda t, te, *_: (te[t], 0, 0)),
            pl.BlockSpec(memory_space=pl.ANY),
            pl.BlockSpec((1, 1, H), lambda t, te, *_: (te[t], 0, 0)),
            pl.BlockSpec((BLK, 1), lambda t, *_: (t, 0)),
        ],
        out_specs=pl.BlockSpec((BLK, H), lambda t, *_: (t, 0)),
        scratch_shapes=[
            pltpu.VMEM((2, H, 2 * I), jnp.float32),
            pltpu.VMEM((2, I, H), jnp.float32),
            pltpu.SemaphoreType.DMA((2,)),
            pltpu.SemaphoreType.DMA((2,)),
        ],
    )
    return pl.pallas_call(
        _expert_body,
        grid_spec=grid_spec,
        out_shape=jax.ShapeDtypeStruct((PAD, H), jnp.float32),
        compiler_params=pltpu.CompilerParams(
            dimension_semantics=("arbitrary",)),
    )(tile_expert, tile_valid, fi, sl, nx, isu,
      xs, w1, b1.reshape(E, 1, 2 * I), w2, b2.reshape(E, 1, H), srw)


def _shared_body(x_ref, w1_ref, b1_ref, w2_ref, b2_ref, o_ref):
    x = x_ref[...]
    h = jnp.dot(x, w1_ref[...], preferred_element_type=jnp.float32)
    h = h + b1_ref[...]
    a = h[:, :I]
    b = h[:, I:]
    hh = (a * jax.nn.sigmoid(a)) * b
    o = jnp.dot(hh, w2_ref[...], preferred_element_type=jnp.float32)
    o_ref[...] = o + b2_ref[...]


def _run_shared(x, sw1, sb1, sw2, sb2):
    nblk = S // BLK
    return pl.pallas_call(
        _shared_body,
        grid=(nblk,),
        in_specs=[
            pl.BlockSpec((BLK, H), lambda t: (t, 0)),
            pl.BlockSpec((H, 2 * I), lambda t: (0, 0)),
            pl.BlockSpec((1, 2 * I), lambda t: (0, 0)),
            pl.BlockSpec((I, H), lambda t: (0, 0)),
            pl.BlockSpec((1, H), lambda t: (0, 0)),
        ],
        out_specs=pl.BlockSpec((BLK, H), lambda t: (t, 0)),
        out_shape=jax.ShapeDtypeStruct((S, H), jnp.float32),
        compiler_params=pltpu.CompilerParams(
            dimension_semantics=("arbitrary",)),
    )(x, sw1, sb1.reshape(1, 2 * I), sw2, sb2.reshape(1, H))


_RR = 32
_RL = 128


def _route_body(idx_ref, pos_ref, te_ref, tv_ref, fi_ref, sl_ref,
                nx_ref, is_ref):
    idx = idx_ref[...]
    ii = lax.broadcasted_iota(jnp.int32, (_RL, _RL), 0)
    jj = lax.broadcasted_iota(jnp.int32, (_RL, _RL), 1)
    upp = (ii <= jj).astype(jnp.float32)
    i2 = lax.broadcasted_iota(jnp.int32, (_RR, _RR), 0)
    j2 = lax.broadcasted_iota(jnp.int32, (_RR, _RR), 1)
    lstrict = (j2 < i2).astype(jnp.float32)

    masks = []
    rexs = []
    sizes = []
    for e in range(E):
        m = (idx == e).astype(jnp.float32)
        cr = jnp.dot(m, upp, preferred_element_type=jnp.float32)
        tot = cr[:, _RL - 1:_RL]
        prior = jnp.dot(lstrict, tot, preferred_element_type=jnp.float32)
        rexs.append(cr - m + prior)
        masks.append(m)
        sizes.append(jnp.sum(tot).astype(jnp.int32))

    tile_cum = []
    c = jnp.int32(0)
    aligned = []
    for e in range(E):
        nt = (sizes[e] + BLK - 1) // BLK
        aligned.append((c * BLK).astype(jnp.float32))
        c = c + nt
        tile_cum.append(c)

    pos = jnp.zeros((_RR, _RL), jnp.float32)
    for e in range(E):
        pos = pos + masks[e] * (rexs[e] + aligned[e])
    pos_ref[...] = pos.astype(jnp.int32)

    ti = lax.broadcasted_iota(jnp.int32, (1, _RL), 1)
    te = jnp.zeros((1, _RL), jnp.int32)
    for e in range(E):
        te = te + (tile_cum[e] <= ti).astype(jnp.int32)
    te = jnp.minimum(te, E - 1)
    te_ref[...] = te
    tv_ref[...] = (ti < tile_cum[E - 1]).astype(jnp.int32)

    nts = [(sizes[e] + BLK - 1) // BLK for e in range(E)]
    starts = [tile_cum[e] - nts[e] for e in range(E)]
    nextof = []
    carry = jnp.int32(-1)
    for e in range(E - 1, -1, -1):
        nextof.append(carry)
        carry = jnp.where(nts[e] > 0, jnp.int32(e), carry)
    nextof = nextof[::-1]
    fi = jnp.zeros((1, _RL), jnp.int32)
    nx = jnp.zeros((1, _RL), jnp.int32)
    ridx = jnp.zeros((1, _RL), jnp.int32)
    for e in range(E):
        nonempty = (nts[e] > 0).astype(jnp.int32)
        fi = fi + (ti == starts[e]).astype(jnp.int32) * nonempty
        in_run = ((ti >= starts[e]) & (ti < tile_cum[e])).astype(jnp.int32)
        nx = nx + in_run * nextof[e]
        ridx = ridx + ((starts[e] <= ti).astype(jnp.int32) * nonempty)
    sl = jnp.where(ridx > 0, (ridx - 1) % 2, 0)
    isu = fi * (nx > te).astype(jnp.int32)
    fi_ref[...] = fi
    sl_ref[...] = sl
    nx_ref[...] = jnp.maximum(nx, 0)
    is_ref[...] = isu


def _routing(flat_idx, flat_w):
    pos2, te, tv, fi, sl, nx, isu = pl.pallas_call(
        _route_body,
        out_shape=(
            jax.ShapeDtypeStruct((_RR, _RL), jnp.int32),
            jax.ShapeDtypeStruct((1, _RL), jnp.int32),
            jax.ShapeDtypeStruct((1, _RL), jnp.int32),
            jax.ShapeDtypeStruct((1, _RL), jnp.int32),
            jax.ShapeDtypeStruct((1, _RL), jnp.int32),
            jax.ShapeDtypeStruct((1, _RL), jnp.int32),
            jax.ShapeDtypeStruct((1, _RL), jnp.int32),
        ),
    )(flat_idx.reshape(_RR, _RL))
    pos = pos2.reshape(NR)
    tile_expert = te.reshape(_RL)
    tile_valid = tv.reshape(_RL)
    sched = (fi.reshape(_RL), sl.reshape(_RL), nx.reshape(_RL),
             isu.reshape(_RL))

    r_ar = jnp.arange(NR, dtype=jnp.int32)
    tok_src = (jnp.arange(PAD, dtype=jnp.int32) % S).at[pos].set(r_ar // K)
    srw = jnp.zeros((PAD,), jnp.float32).at[pos].set(flat_w)
    p0 = pos[0::2]
    p1 = pos[1::2]
    return tile_expert, tile_valid, sched, tok_src, srw, p0, p1


def kernel(hidden_states, topk_weight, topk_idx, w1, b1, w2, b2,
           sw1, sb1, sw2, sb2):
    orig_shape = hidden_states.shape
    x = hidden_states.reshape(S, H)
    flat_idx = topk_idx.reshape(NR).astype(jnp.int32)
    flat_w = topk_weight.reshape(NR)
    tile_expert, tile_valid, sched, tok_src, srw, p0, p1 = _routing(
        flat_idx, flat_w)

    sc_gather, sc_combine = _build_sc_kernels()
    sh = _run_shared(x, sw1, sb1, sw2, sb2)
    xs = sc_gather(x, tok_src)
    ys = _run_experts(tile_expert, tile_valid, sched, xs, w1, b1, w2, b2,
                      srw.reshape(PAD, 1))
    out = sc_combine(ys, p0, p1, sh)
    return out.reshape(orig_shape)


# speedup vs baseline: 1.1885x; 1.0095x over previous
import functools

import jax
import jax.numpy as jnp
from jax import lax
from jax.experimental import pallas as pl
from jax.experimental.pallas import tpu as pltpu
from jax.experimental.pallas import tpu_sc as plsc

E = 16
H = 1024
I = 1024
S = 2048
K = 2
NR = S * K
BLK = 256
MAX_TILES = NR // BLK + E
PAD = MAX_TILES * BLK

NC, NS, L = 2, 16, 16
NW = NC * NS
G_CH = 32
T_CH = 16

@functools.lru_cache(maxsize=None)
def _build_sc_kernels():
    mesh = plsc.VectorSubcoreMesh(
        core_axis_name="c", subcore_axis_name="s",
        num_cores=NC, num_subcores=NS)

    RPW = PAD // NW
    NCH = RPW // G_CH

    @functools.partial(
        pl.kernel,
        out_type=jax.ShapeDtypeStruct((PAD, H), jnp.float32),
        mesh=mesh,
        scratch_types=[
            pltpu.VMEM((RPW,), jnp.int32),
            pltpu.VMEM((G_CH, H), jnp.float32),
            pltpu.VMEM((G_CH, H), jnp.float32),
            pltpu.VMEM((G_CH, H), jnp.float32),
            pltpu.SemaphoreType.DMA,
            pltpu.SemaphoreType.DMA,
            pltpu.SemaphoreType.DMA,
            pltpu.SemaphoreType.DMA,
            pltpu.SemaphoreType.DMA,
            pltpu.SemaphoreType.DMA,
        ],
    )
    def sc_gather(x_hbm, tok_hbm, xs_hbm, idx_v, b0, b1, b2,
                  g0, g1, g2, s0, s1, s2):
        wid = lax.axis_index("s") * NC + lax.axis_index("c")
        base = wid * RPW
        bufs = (b0, b1, b2)
        gsems = (g0, g1, g2)
        ssems = (s0, s1, s2)
        pltpu.sync_copy(tok_hbm.at[pl.ds(base, RPW)], idx_v)

        def gfire(c):
            return pltpu.async_copy(
                x_hbm.at[idx_v.at[pl.ds(c * G_CH, G_CH)]],
                bufs[c % 3], gsems[c % 3])

        def sfire(c):
            return pltpu.async_copy(
                bufs[c % 3], xs_hbm.at[pl.ds(base + c * G_CH, G_CH)],
                ssems[c % 3])

        g = [None] * NCH
        s = [None] * NCH
        for c in range(min(3, NCH)):
            g[c] = gfire(c)
        for c in range(NCH):
            g[c].wait()
            s[c] = sfire(c)
            if c + 3 < NCH:
                s[c].wait()
                g[c + 3] = gfire(c + 3)
        for c in range(max(NCH - 3, 0), NCH):
            s[c].wait()

    TPW = S // NW
    TNCH = TPW // T_CH

    @functools.partial(
        pl.kernel,
        out_type=jax.ShapeDtypeStruct((S, H), jnp.float32),
        mesh=mesh,
        scratch_types=[
            pltpu.VMEM((TPW,), jnp.int32),
            pltpu.VMEM((TPW,), jnp.int32),
            pltpu.VMEM((T_CH, H), jnp.float32),
            pltpu.VMEM((T_CH, H), jnp.float32),
            pltpu.VMEM((T_CH, H), jnp.float32),
            pltpu.VMEM((T_CH, H), jnp.float32),
            pltpu.VMEM((T_CH, H), jnp.float32),
            pltpu.VMEM((T_CH, H), jnp.float32),
            pltpu.SemaphoreType.DMA,
            pltpu.SemaphoreType.DMA,
            pltpu.SemaphoreType.DMA,
            pltpu.SemaphoreType.DMA,
        ],
    )
    def sc_combine(ys_hbm, p0_hbm, p1_hbm, sh_hbm, out_hbm,
                   i0_v, i1_v, y0a, y1a, sha, y0b, y1b, shb,
                   ga, gb, sa, sb):
        wid = lax.axis_index("s") * NC + lax.axis_index("c")
        base = wid * TPW
        y0s = (y0a, y0b)
        y1s = (y1a, y1b)
        shs = (sha, shb)
        gsems = (ga, gb)
        ssems = (sa, sb)
        pltpu.sync_copy(p0_hbm.at[pl.ds(base, TPW)], i0_v)
        pltpu.sync_copy(p1_hbm.at[pl.ds(base, TPW)], i1_v)

        def fire_in(c):
            sl = c % 2
            return (
                pltpu.async_copy(
                    ys_hbm.at[i0_v.at[pl.ds(c * T_CH, T_CH)]],
                    y0s[sl], gsems[sl]),
                pltpu.async_copy(
                    ys_hbm.at[i1_v.at[pl.ds(c * T_CH, T_CH)]],
                    y1s[sl], gsems[sl]),
                pltpu.async_copy(
                    sh_hbm.at[pl.ds(base + c * T_CH, T_CH)],
                    shs[sl], gsems[sl]),
            )

        def fire_out(c):
            sl = c % 2
            return pltpu.async_copy(
                y0s[sl], out_hbm.at[pl.ds(base + c * T_CH, T_CH)], ssems[sl])

        ins = [None] * TNCH
        outs = [None] * TNCH
        for c in range(min(2, TNCH)):
            ins[c] = fire_in(c)
        for c in range(TNCH):
            sl = c % 2
            for cp in ins[c]:
                cp.wait()
            y0r, y1r, shr = y0s[sl], y1s[sl], shs[sl]

            def row_body(r, _):
                for cc in range(H // L):
                    sli = pl.ds(cc * L, L)
                    y0r[r, sli] = y0r[r, sli] + y1r[r, sli] + shr[r, sli]
                return 0

            lax.fori_loop(0, T_CH, row_body, 0)
            outs[c] = fire_out(c)
            if c + 2 < TNCH:
                outs[c].wait()
                ins[c + 2] = fire_in(c + 2)
        for c in range(max(TNCH - 2, 0), TNCH):
            outs[c].wait()

    return sc_gather, sc_combine


def _expert_body(te_ref, tv_ref, fi_ref, sl_ref, nx_ref, is_ref,
                 x_ref, w1_any, b1_ref, w2_any, b2_ref, sw_ref, y_ref,
                 w1b, w2b, sem1, sem2):
    t = pl.program_id(0)
    slot = sl_ref[t]
    e = te_ref[t]

    def w1_copy(src_e, dst_slot):
        return pltpu.make_async_copy(
            w1_any.at[pl.ds(src_e, 1)], w1b.at[pl.ds(dst_slot, 1)],
            sem1.at[dst_slot])

    def w2_copy(src_e, dst_slot):
        return pltpu.make_async_copy(
            w2_any.at[pl.ds(src_e, 1)], w2b.at[pl.ds(dst_slot, 1)],
            sem2.at[dst_slot])

    @pl.when(t == 0)
    def _():
        w1_copy(e, slot).start()
        w2_copy(e, slot).start()

    @pl.when((t == 0) & (nx_ref[0] > te_ref[0]))
    def _():
        w1_copy(nx_ref[0], (slot + 1) % 3).start()
        w2_copy(nx_ref[0], (slot + 1) % 3).start()

    @pl.when(fi_ref[t] == 1)
    def _():
        w1_copy(e, slot).wait()
        w2_copy(e, slot).wait()

    @pl.when((fi_ref[t] == 1) & (is_ref[t] > e))
    def _():
        w1_copy(is_ref[t], (slot + 2) % 3).start()
        w2_copy(is_ref[t], (slot + 2) % 3).start()

    @pl.when(tv_ref[t] > 0)
    def _():
        x = x_ref[...]
        h = jnp.dot(x, w1b[slot], preferred_element_type=jnp.float32)
        h = h + b1_ref[0]
        a = h[:, :I]
        b = h[:, I:]
        hh = (a * jax.nn.sigmoid(a)) * b
        y = jnp.dot(hh, w2b[slot], preferred_element_type=jnp.float32)
        y = y + b2_ref[0]
        y_ref[...] = y * sw_ref[...]


def _run_experts(tile_expert, tile_valid, sched, xs, w1, b1, w2, b2, srw):
    fi, sl, nx, isu = sched
    grid_spec = pltpu.PrefetchScalarGridSpec(
        num_scalar_prefetch=6,
        grid=(MAX_TILES,),
        in_specs=[
            pl.BlockSpec((BLK, H), lambda t, *_: (t, 0)),
            pl.BlockSpec(memory_space=pl.ANY),
            pl.BlockSpec((1, 1, 2 * I), lambda t, te, *_: (te[t], 0, 0)),
            pl.BlockSpec(memory_space=pl.ANY),
            pl.BlockSpec((1, 1, H), lambda t, te, *_: (te[t], 0, 0)),
            pl.BlockSpec((BLK, 1), lambda t, *_: (t, 0)),
        ],
        out_specs=pl.BlockSpec((BLK, H), lambda t, *_: (t, 0)),
        scratch_shapes=[
            pltpu.VMEM((3, H, 2 * I), jnp.float32),
            pltpu.VMEM((3, I, H), jnp.float32),
            pltpu.SemaphoreType.DMA((3,)),
            pltpu.SemaphoreType.DMA((3,)),
        ],
    )
    return pl.pallas_call(
        _expert_body,
        grid_spec=grid_spec,
        out_shape=jax.ShapeDtypeStruct((PAD, H), jnp.float32),
        compiler_params=pltpu.CompilerParams(
            dimension_semantics=("arbitrary",)),
    )(tile_expert, tile_valid, fi, sl, nx, isu,
      xs, w1, b1.reshape(E, 1, 2 * I), w2, b2.reshape(E, 1, H), srw)


def _shared_body(x_ref, w1_ref, b1_ref, w2_ref, b2_ref, o_ref):
    x = x_ref[...]
    h = jnp.dot(x, w1_ref[...], preferred_element_type=jnp.float32)
    h = h + b1_ref[...]
    a = h[:, :I]
    b = h[:, I:]
    hh = (a * jax.nn.sigmoid(a)) * b
    o = jnp.dot(hh, w2_ref[...], preferred_element_type=jnp.float32)
    o_ref[...] = o + b2_ref[...]


def _run_shared(x, sw1, sb1, sw2, sb2):
    nblk = S // BLK
    return pl.pallas_call(
        _shared_body,
        grid=(nblk,),
        in_specs=[
            pl.BlockSpec((BLK, H), lambda t: (t, 0)),
            pl.BlockSpec((H, 2 * I), lambda t: (0, 0)),
            pl.BlockSpec((1, 2 * I), lambda t: (0, 0)),
            pl.BlockSpec((I, H), lambda t: (0, 0)),
            pl.BlockSpec((1, H), lambda t: (0, 0)),
        ],
        out_specs=pl.BlockSpec((BLK, H), lambda t: (t, 0)),
        out_shape=jax.ShapeDtypeStruct((S, H), jnp.float32),
        compiler_params=pltpu.CompilerParams(
            dimension_semantics=("arbitrary",)),
    )(x, sw1, sb1.reshape(1, 2 * I), sw2, sb2.reshape(1, H))


_RR = 32
_RL = 128


def _route_body(idx_ref, pos_ref, te_ref, tv_ref, fi_ref, sl_ref,
                nx_ref, is_ref):
    idx = idx_ref[...]
    ii = lax.broadcasted_iota(jnp.int32, (_RL, _RL), 0)
    jj = lax.broadcasted_iota(jnp.int32, (_RL, _RL), 1)
    upp = (ii <= jj).astype(jnp.float32)
    i2 = lax.broadcasted_iota(jnp.int32, (_RR, _RR), 0)
    j2 = lax.broadcasted_iota(jnp.int32, (_RR, _RR), 1)
    lstrict = (j2 < i2).astype(jnp.float32)

    masks = []
    rexs = []
    sizes = []
    for e in range(E):
        m = (idx == e).astype(jnp.float32)
        cr = jnp.dot(m, upp, preferred_element_type=jnp.float32)
        tot = cr[:, _RL - 1:_RL]
        prior = jnp.dot(lstrict, tot, preferred_element_type=jnp.float32)
        rexs.append(cr - m + prior)
        masks.append(m)
        sizes.append(jnp.sum(tot).astype(jnp.int32))

    tile_cum = []
    c = jnp.int32(0)
    aligned = []
    for e in range(E):
        nt = (sizes[e] + BLK - 1) // BLK
        aligned.append((c * BLK).astype(jnp.float32))
        c = c + nt
        tile_cum.append(c)

    pos = jnp.zeros((_RR, _RL), jnp.float32)
    for e in range(E):
        pos = pos + masks[e] * (rexs[e] + aligned[e])
    pos_ref[...] = pos.astype(jnp.int32)

    ti = lax.broadcasted_iota(jnp.int32, (1, _RL), 1)
    te = jnp.zeros((1, _RL), jnp.int32)
    for e in range(E):
        te = te + (tile_cum[e] <= ti).astype(jnp.int32)
    te = jnp.minimum(te, E - 1)
    te_ref[...] = te
    tv_ref[...] = (ti < tile_cum[E - 1]).astype(jnp.int32)

    nts = [(sizes[e] + BLK - 1) // BLK for e in range(E)]
    starts = [tile_cum[e] - nts[e] for e in range(E)]
    nextof = []
    next2of = []
    ca = jnp.int32(-1)
    cb = jnp.int32(-1)
    for e in range(E - 1, -1, -1):
        nextof.append(ca)
        next2of.append(cb)
        nonempty = nts[e] > 0
        cb = jnp.where(nonempty, ca, cb)
        ca = jnp.where(nonempty, jnp.int32(e), ca)
    nextof = nextof[::-1]
    next2of = next2of[::-1]
    fi = jnp.zeros((1, _RL), jnp.int32)
    nx = jnp.zeros((1, _RL), jnp.int32)
    nx2 = jnp.zeros((1, _RL), jnp.int32)
    ridx = jnp.zeros((1, _RL), jnp.int32)
    for e in range(E):
        nonempty = (nts[e] > 0).astype(jnp.int32)
        fi = fi + (ti == starts[e]).astype(jnp.int32) * nonempty
        in_run = ((ti >= starts[e]) & (ti < tile_cum[e])).astype(jnp.int32)
        nx = nx + in_run * nextof[e]
        nx2 = nx2 + in_run * next2of[e]
        ridx = ridx + ((starts[e] <= ti).astype(jnp.int32) * nonempty)
    sl = jnp.where(ridx > 0, (ridx - 1) % 3, 0)
    fi_ref[...] = fi
    sl_ref[...] = sl
    nx_ref[...] = jnp.maximum(nx, 0)
    is_ref[...] = jnp.maximum(nx2, 0)


def _routing(flat_idx, flat_w):
    pos2, te, tv, fi, sl, nx, isu = pl.pallas_call(
        _route_body,
        out_shape=(
            jax.ShapeDtypeStruct((_RR, _RL), jnp.int32),
            jax.ShapeDtypeStruct((1, _RL), jnp.int32),
            jax.ShapeDtypeStruct((1, _RL), jnp.int32),
            jax.ShapeDtypeStruct((1, _RL), jnp.int32),
            jax.ShapeDtypeStruct((1, _RL), jnp.int32),
            jax.ShapeDtypeStruct((1, _RL), jnp.int32),
            jax.ShapeDtypeStruct((1, _RL), jnp.int32),
        ),
    )(flat_idx.reshape(_RR, _RL))
    pos = pos2.reshape(NR)
    tile_expert = te.reshape(_RL)
    tile_valid = tv.reshape(_RL)
    sched = (fi.reshape(_RL), sl.reshape(_RL), nx.reshape(_RL),
             isu.reshape(_RL))

    r_ar = jnp.arange(NR, dtype=jnp.int32)
    tok_src = (jnp.arange(PAD, dtype=jnp.int32) % S).at[pos].set(r_ar // K)
    srw = jnp.zeros((PAD,), jnp.float32).at[pos].set(flat_w)
    p0 = pos[0::2]
    p1 = pos[1::2]
    return tile_expert, tile_valid, sched, tok_src, srw, p0, p1


def kernel(hidden_states, topk_weight, topk_idx, w1, b1, w2, b2,
           sw1, sb1, sw2, sb2):
    orig_shape = hidden_states.shape
    x = hidden_states.reshape(S, H)
    flat_idx = topk_idx.reshape(NR).astype(jnp.int32)
    flat_w = topk_weight.reshape(NR)
    tile_expert, tile_valid, sched, tok_src, srw, p0, p1 = _routing(
        flat_idx, flat_w)

    sc_gather, sc_combine = _build_sc_kernels()
    sh = _run_shared(x, sw1, sb1, sw2, sb2)
    xs = sc_gather(x, tok_src)
    ys = _run_experts(tile_expert, tile_valid, sched, xs, w1, b1, w2, b2,
                      srw.reshape(PAD, 1))
    out = sc_combine(ys, p0, p1, sh)
    return out.reshape(orig_shape)
